# baseline (speedup 1.0000x reference)
import functools

import jax
import jax.numpy as jnp
from jax import lax
from jax.experimental import pallas as pl
from jax.experimental.pallas import tpu as pltpu
from jax.experimental.pallas import tpu_sc as plsc

_VOCAB = 100000
_D = 64
_B = 1024

_NC = 2
_NS = 16
_NW = _NC * _NS
_BPW = _B // _NW


@functools.cache
def _make_sc_gather():
    mesh = plsc.VectorSubcoreMesh(core_axis_name="c", subcore_axis_name="s")

    @functools.partial(
        pl.kernel,
        mesh=mesh,
        compiler_params=pltpu.CompilerParams(use_tc_tiling_on_sc=False),
        out_type=jax.ShapeDtypeStruct((_B, _D), jnp.float32),
        scratch_types=[
            pltpu.VMEM((_BPW,), jnp.int32),
            pltpu.VMEM((_BPW, _D), jnp.float32),
            pltpu.SemaphoreType.DMA,
        ],
    )
    def _sc_gather(idx_hbm, table_hbm, out_hbm, idx_v, rows_v, sem):
        wid = lax.axis_index("s") * _NC + lax.axis_index("c")
        base = wid * _BPW
        pltpu.sync_copy(idx_hbm.at[pl.ds(base, _BPW)], idx_v)
        pltpu.async_copy(table_hbm.at[idx_v], rows_v, sem).wait()
        pltpu.sync_copy(rows_v, out_hbm.at[pl.ds(base, _BPW)])

    return _sc_gather


_VBLK = 1024
_NFULL = _VOCAB // _VBLK
_NBUF = 8


def _dot_bias(e_ref, w_ref, b_ref):
    return (
        lax.dot_general(
            e_ref[...], w_ref[...],
            (((1,), (1,)), ((), ())),
            preferred_element_type=jnp.float32,
        )
        + b_ref[...]
    )


def _mm_body(e_ref, w_ref, b_ref, o_ref, acc_ref, sems):
    j = pl.program_id(0)
    slot = lax.rem(j, _NBUF)

    def copy(step, s):
        return pltpu.make_async_copy(
            acc_ref.at[s],
            o_ref.at[:, pl.ds(step * _VBLK, _VBLK)],
            sems.at[s],
        )

    @pl.when(j >= _NBUF)
    def _():
        copy(j - _NBUF, slot).wait()

    acc_ref[slot] = jnp.broadcast_to(b_ref[...], (_B, _VBLK))
    copy(j, slot).start()

    @pl.when(j == _NFULL - 1)
    def _():
        for jj in range(max(_NFULL - _NBUF, 0), _NFULL):
            copy(jj, jj % _NBUF).wait()


def _tail_body(e_ref, w_ref, b_ref, alias_ref, o_ref):
    del alias_ref
    o_ref[...] = _dot_bias(e_ref, w_ref, b_ref)


def _tc_project(embeds, W, b2d):
    main = pl.pallas_call(
        _mm_body,
        grid=(_NFULL,),
        in_specs=[
            pl.BlockSpec((_B, _D), lambda j: (0, 0)),
            pl.BlockSpec((_VBLK, _D), lambda j: (j, 0)),
            pl.BlockSpec((1, _VBLK), lambda j: (0, j)),
        ],
        out_specs=pl.BlockSpec(memory_space=pl.ANY),
        out_shape=jax.ShapeDtypeStruct((_B, _VOCAB), jnp.float32),
        scratch_shapes=[
            pltpu.VMEM((_NBUF, _B, _VBLK), jnp.float32),
            pltpu.SemaphoreType.DMA((_NBUF,)),
        ],
    )(embeds, W, b2d)

    return pl.pallas_call(
        _tail_body,
        grid=(1,),
        in_specs=[
            pl.BlockSpec((_B, _D), lambda j: (0, 0)),
            pl.BlockSpec((_VBLK, _D), lambda j: (_NFULL, 0)),
            pl.BlockSpec((1, _VBLK), lambda j: (0, _NFULL)),
            pl.BlockSpec(memory_space=pl.ANY),
        ],
        out_specs=pl.BlockSpec((_B, _VBLK), lambda j: (0, _NFULL)),
        out_shape=jax.ShapeDtypeStruct((_B, _VOCAB), jnp.float32),
        input_output_aliases={3: 0},
    )(embeds, W, b2d, main)


def kernel(x, table, W, b):
    idx = x.astype(jnp.int32)
    embeds = _make_sc_gather()(idx, table)
    return _tc_project(embeds, W, b.reshape(1, _VOCAB))


# speedup vs baseline: 1.0586x; 1.0586x over previous
import functools

import jax
import jax.numpy as jnp
from jax import lax
from jax.experimental import pallas as pl
from jax.experimental.pallas import tpu as pltpu
from jax.experimental.pallas import tpu_sc as plsc

_VOCAB = 100000
_D = 64
_B = 1024

_NC = 2
_NS = 16
_NW = _NC * _NS
_BPW = _B // _NW


@functools.cache
def _make_sc_gather():
    mesh = plsc.VectorSubcoreMesh(core_axis_name="c", subcore_axis_name="s")

    @functools.partial(
        pl.kernel,
        mesh=mesh,
        compiler_params=pltpu.CompilerParams(use_tc_tiling_on_sc=False),
        out_type=jax.ShapeDtypeStruct((_B, _D), jnp.float32),
        scratch_types=[
            pltpu.VMEM((_BPW,), jnp.int32),
            pltpu.VMEM((_BPW, _D), jnp.float32),
            pltpu.SemaphoreType.DMA,
        ],
    )
    def _sc_gather(idx_hbm, table_hbm, out_hbm, idx_v, rows_v, sem):
        wid = lax.axis_index("s") * _NC + lax.axis_index("c")
        base = wid * _BPW
        pltpu.sync_copy(idx_hbm.at[pl.ds(base, _BPW)], idx_v)
        pltpu.async_copy(table_hbm.at[idx_v], rows_v, sem).wait()
        pltpu.sync_copy(rows_v, out_hbm.at[pl.ds(base, _BPW)])

    return _sc_gather


_MBLK = 16
_NSTEP = _B // _MBLK


def _mm_body(e_ref, wt_ref, b_ref, o_ref):
    o_ref[...] = (
        jnp.dot(e_ref[...], wt_ref[...], preferred_element_type=jnp.float32)
        + b_ref[...]
    )


def _tc_project(embeds, Wt, b2d):
    return pl.pallas_call(
        _mm_body,
        grid=(_NSTEP,),
        in_specs=[
            pl.BlockSpec((_MBLK, _D), lambda j: (j, 0)),
            pl.BlockSpec((_D, _VOCAB), lambda j: (0, 0)),
            pl.BlockSpec((1, _VOCAB), lambda j: (0, 0)),
        ],
        out_specs=pl.BlockSpec((_MBLK, _VOCAB), lambda j: (j, 0)),
        out_shape=jax.ShapeDtypeStruct((_B, _VOCAB), jnp.float32),
    )(embeds, Wt, b2d)


def kernel(x, table, W, b):
    idx = x.astype(jnp.int32)
    embeds = _make_sc_gather()(idx, table)
    return _tc_project(embeds, jnp.swapaxes(W, 0, 1), b.reshape(1, _VOCAB))


# speedup vs baseline: 1.0985x; 1.0377x over previous
import functools

import jax
import jax.numpy as jnp
from jax import lax
from jax.experimental import pallas as pl
from jax.experimental.pallas import tpu as pltpu
from jax.experimental.pallas import tpu_sc as plsc

_VOCAB = 100000
_D = 64
_B = 1024

_NC = 2
_NS = 16
_NW = _NC * _NS
_BPW = _B // _NW


@functools.cache
def _make_sc_gather():
    mesh = plsc.VectorSubcoreMesh(core_axis_name="c", subcore_axis_name="s")

    @functools.partial(
        pl.kernel,
        mesh=mesh,
        compiler_params=pltpu.CompilerParams(use_tc_tiling_on_sc=False),
        out_type=jax.ShapeDtypeStruct((_B, _D), jnp.float32),
        scratch_types=[
            pltpu.VMEM((_BPW,), jnp.int32),
            pltpu.VMEM((_BPW, _D), jnp.float32),
            pltpu.SemaphoreType.DMA,
        ],
    )
    def _sc_gather(idx_hbm, table_hbm, out_hbm, idx_v, rows_v, sem):
        wid = lax.axis_index("s") * _NC + lax.axis_index("c")
        base = wid * _BPW
        pltpu.sync_copy(idx_hbm.at[pl.ds(base, _BPW)], idx_v)
        pltpu.async_copy(table_hbm.at[idx_v], rows_v, sem).wait()
        pltpu.sync_copy(rows_v, out_hbm.at[pl.ds(base, _BPW)])

    return _sc_gather


_MBLK = 16
_NSTEP = _B // _MBLK
_NBUF = 4


def _mm_body(e_ref, wt_ref, b_ref, o_ref, acc_ref, sems):
    j = pl.program_id(0)
    slot = lax.rem(j, _NBUF)

    def copy(step, s):
        return pltpu.make_async_copy(
            acc_ref.at[s],
            o_ref.at[pl.ds(step * _MBLK, _MBLK)],
            sems.at[s],
        )

    @pl.when(j >= _NBUF)
    def _():
        copy(j - _NBUF, slot).wait()

    acc_ref[slot] = (
        jnp.dot(e_ref[...], wt_ref[...], preferred_element_type=jnp.float32)
        + b_ref[...]
    )

    @pl.when(lax.rem(j, 2) == 0)
    def _():
        copy(j, slot).start(priority=0)

    @pl.when(lax.rem(j, 2) == 1)
    def _():
        copy(j, slot).start(priority=1)

    @pl.when(j == _NSTEP - 1)
    def _():
        for jj in range(_NSTEP - _NBUF, _NSTEP):
            copy(jj, jj % _NBUF).wait()


def _tc_project(embeds, Wt, b2d):
    return pl.pallas_call(
        _mm_body,
        grid=(_NSTEP,),
        in_specs=[
            pl.BlockSpec((_MBLK, _D), lambda j: (j, 0)),
            pl.BlockSpec((_D, _VOCAB), lambda j: (0, 0)),
            pl.BlockSpec((1, _VOCAB), lambda j: (0, 0)),
        ],
        out_specs=pl.BlockSpec(memory_space=pl.ANY),
        out_shape=jax.ShapeDtypeStruct((_B, _VOCAB), jnp.float32),
        scratch_shapes=[
            pltpu.VMEM((_NBUF, _MBLK, _VOCAB), jnp.float32),
            pltpu.SemaphoreType.DMA((_NBUF,)),
        ],
    )(embeds, Wt, b2d)


def kernel(x, table, W, b):
    idx = x.astype(jnp.int32)
    embeds = _make_sc_gather()(idx, table)
    return _tc_project(embeds, jnp.swapaxes(W, 0, 1), b.reshape(1, _VOCAB))
